# baseline (speedup 1.0000x reference)
import jax
import jax.numpy as jnp
from jax.experimental import pallas as pl
from jax.experimental.pallas import tpu as pltpu

_TILE = 128
_NSTREAM = 16
_STEP = _TILE * _NSTREAM


def _route_tile(x_tile, w, bias):
    scores = jax.lax.dot_general(
        x_tile, w,
        (((1,), (1,)), ((), ())),
        preferred_element_type=jnp.float32,
    )
    scores = scores + bias
    m = jnp.max(scores, axis=-1, keepdims=True)
    e = jnp.exp(scores - m)
    s = e / jnp.sum(e, axis=-1, keepdims=True)
    n = s.shape[-1]
    iota = jax.lax.broadcasted_iota(jnp.int32, s.shape, 1)
    m1 = jnp.max(s, axis=-1, keepdims=True)
    i1 = jnp.min(jnp.where(s == m1, iota, n), axis=-1, keepdims=True)
    s2 = jnp.where(iota == i1, -jnp.inf, s)
    m2 = jnp.max(s2, axis=-1, keepdims=True)
    i2 = jnp.min(jnp.where(s2 == m2, iota, n), axis=-1, keepdims=True)
    w2 = jnp.concatenate([m1, m2], axis=1)
    i2c = jnp.concatenate([i1, i2], axis=1)
    return w2.T, i2c.T


def _router_body(*refs):
    x_refs = refs[:_NSTREAM]
    w_ref, b_ref = refs[_NSTREAM], refs[_NSTREAM + 1]
    w_out_ref, i_out_ref = refs[_NSTREAM + 2], refs[_NSTREAM + 3]
    w = w_ref[...]
    bias = b_ref[...]
    for k in range(0, _NSTREAM, 8):
        x_pair = jnp.concatenate([r[...] for r in x_refs[k:k + 8]], axis=0)
        wk, ik = _route_tile(x_pair, w, bias)
        w_out_ref[:, pl.ds(k * _TILE, 8 * _TILE)] = wk
        i_out_ref[:, pl.ds(k * _TILE, 8 * _TILE)] = ik


@jax.jit
def kernel(x, W, b):
    ntok, dim = x.shape
    nexp = W.shape[0]
    grid = (ntok // _STEP,)

    weights_t, idx_t = pl.pallas_call(
        _router_body,
        grid=grid,
        in_specs=[
            pl.BlockSpec((_TILE, dim), (lambda i, k=k: (i * _NSTREAM + k, 0)))
            for k in range(_NSTREAM)
        ] + [
            pl.BlockSpec((nexp, dim), lambda i: (0, 0)),
            pl.BlockSpec((nexp,), lambda i: (0,)),
        ],
        out_specs=[
            pl.BlockSpec((2, _STEP), lambda i: (0, i)),
            pl.BlockSpec((2, _STEP), lambda i: (0, i)),
        ],
        out_shape=[
            jax.ShapeDtypeStruct((2, ntok), jnp.float32),
            jax.ShapeDtypeStruct((2, ntok), jnp.int32),
        ],
        compiler_params=pltpu.CompilerParams(
            dimension_semantics=("parallel",),
        ),
    )(*([x] * _NSTREAM), W, b)
    return weights_t.T, idx_t.T


# speedup vs baseline: 1.0320x; 1.0320x over previous
import jax
import jax.numpy as jnp
from jax.experimental import pallas as pl
from jax.experimental.pallas import tpu as pltpu

_TILE = 128
_NSTREAM = 16
_STEP = _TILE * _NSTREAM


def _route_tile(x_tile, w, bias):
    scores = jax.lax.dot_general(
        x_tile, w,
        (((1,), (1,)), ((), ())),
        preferred_element_type=jnp.float32,
    )
    scores = scores + bias
    m = jnp.max(scores, axis=-1, keepdims=True)
    e = jnp.exp(scores - m)
    s = e / jnp.sum(e, axis=-1, keepdims=True)
    n = s.shape[-1]
    iota = jax.lax.broadcasted_iota(jnp.int32, s.shape, 1)
    m1 = jnp.max(s, axis=-1, keepdims=True)
    i1 = jnp.min(jnp.where(s == m1, iota, n), axis=-1, keepdims=True)
    s2 = jnp.where(iota == i1, -jnp.inf, s)
    m2 = jnp.max(s2, axis=-1, keepdims=True)
    i2 = jnp.min(jnp.where(s2 == m2, iota, n), axis=-1, keepdims=True)
    w2 = jnp.concatenate([m1, m2], axis=1)
    i2c = jnp.concatenate([i1, i2], axis=1)
    return w2.T, i2c.T


def _router_body(*refs):
    x_refs = refs[:_NSTREAM]
    w_ref, b_ref = refs[_NSTREAM], refs[_NSTREAM + 1]
    w_out_ref, i_out_ref = refs[_NSTREAM + 2], refs[_NSTREAM + 3]
    w = w_ref[...]
    bias = b_ref[...]
    for k in range(0, _NSTREAM, 2):
        x_pair = jnp.concatenate([r[...] for r in x_refs[k:k + 2]], axis=0)
        wk, ik = _route_tile(x_pair, w, bias)
        w_out_ref[:, pl.ds(k * _TILE, 2 * _TILE)] = wk
        i_out_ref[:, pl.ds(k * _TILE, 2 * _TILE)] = ik


@jax.jit
def kernel(x, W, b):
    ntok, dim = x.shape
    nexp = W.shape[0]
    grid = (ntok // _STEP,)

    weights_t, idx_t = pl.pallas_call(
        _router_body,
        grid=grid,
        in_specs=[
            pl.BlockSpec((_TILE, dim), (lambda i, k=k: (i * _NSTREAM + k, 0)))
            for k in range(_NSTREAM)
        ] + [
            pl.BlockSpec((nexp, dim), lambda i: (0, 0)),
            pl.BlockSpec((nexp,), lambda i: (0,)),
        ],
        out_specs=[
            pl.BlockSpec((2, _STEP), lambda i: (0, i)),
            pl.BlockSpec((2, _STEP), lambda i: (0, i)),
        ],
        out_shape=[
            jax.ShapeDtypeStruct((2, ntok), jnp.float32),
            jax.ShapeDtypeStruct((2, ntok), jnp.int32),
        ],
        compiler_params=pltpu.CompilerParams(
            dimension_semantics=("parallel",),
        ),
    )(*([x] * _NSTREAM), W, b)
    return weights_t.T, idx_t.T


# speedup vs baseline: 1.0765x; 1.0431x over previous
import jax
import jax.numpy as jnp
from jax.experimental import pallas as pl
from jax.experimental.pallas import tpu as pltpu

_TILE = 128
_NSTREAM = 16
_GROUP = 4
_STEP = _TILE * _NSTREAM


def _route_tile(x_tile, w, bias):
    scores = jax.lax.dot_general(
        x_tile, w,
        (((1,), (1,)), ((), ())),
        preferred_element_type=jnp.float32,
    )
    scores = scores + bias
    m = jnp.max(scores, axis=-1, keepdims=True)
    e = jnp.exp(scores - m)
    s = e / jnp.sum(e, axis=-1, keepdims=True)
    n = s.shape[-1]
    iota = jax.lax.broadcasted_iota(jnp.int32, s.shape, 1)
    m1 = jnp.max(s, axis=-1, keepdims=True)
    i1 = jnp.min(jnp.where(s == m1, iota, n), axis=-1, keepdims=True)
    s2 = jnp.where(iota == i1, -jnp.inf, s)
    m2 = jnp.max(s2, axis=-1, keepdims=True)
    i2 = jnp.min(jnp.where(s2 == m2, iota, n), axis=-1, keepdims=True)
    w2 = jnp.concatenate([m1, m2], axis=1)
    i2c = jnp.concatenate([i1, i2], axis=1)
    return w2.T, i2c.T


def _router_body(*refs):
    x_refs = refs[:_NSTREAM]
    w_ref, b_ref = refs[_NSTREAM], refs[_NSTREAM + 1]
    w_out_ref, i_out_ref = refs[_NSTREAM + 2], refs[_NSTREAM + 3]
    w = w_ref[...]
    bias = b_ref[...]
    for k in range(0, _NSTREAM, _GROUP):
        x_tile = jnp.concatenate([r[...] for r in x_refs[k:k + _GROUP]],
                                 axis=0)
        wk, ik = _route_tile(x_tile, w, bias)
        w_out_ref[:, pl.ds(k * _TILE, _GROUP * _TILE)] = wk
        i_out_ref[:, pl.ds(k * _TILE, _GROUP * _TILE)] = ik


@jax.jit
def kernel(x, W, b):
    ntok, dim = x.shape
    nexp = W.shape[0]

    weights_t, idx_t = pl.pallas_call(
        _router_body,
        grid=(ntok // _STEP,),
        in_specs=[
            pl.BlockSpec((_TILE, dim), (lambda i, k=k: (i * _NSTREAM + k, 0)))
            for k in range(_NSTREAM)
        ] + [
            pl.BlockSpec((nexp, dim), lambda i: (0, 0)),
            pl.BlockSpec((nexp,), lambda i: (0,)),
        ],
        out_specs=[
            pl.BlockSpec((2, _STEP), lambda i: (0, i)),
            pl.BlockSpec((2, _STEP), lambda i: (0, i)),
        ],
        out_shape=[
            jax.ShapeDtypeStruct((2, ntok), jnp.float32),
            jax.ShapeDtypeStruct((2, ntok), jnp.int32),
        ],
        compiler_params=pltpu.CompilerParams(
            dimension_semantics=("parallel",),
        ),
    )(*([x] * _NSTREAM), W, b)
    return weights_t.T, idx_t.T
